# baseline (device time: 114485 ns/iter reference)
import jax
import jax.numpy as jnp
from jax import lax
from jax.experimental import pallas as pl
from jax.experimental.pallas import tpu as pltpu

N_DEV = 8
BLK = 64


def kernel(x, Wq, K_ext, V_ext, Wo):
    B, Sq_l, Dm = x.shape
    _, _, Hq, Dh = K_ext.shape
    S_g = N_DEV * Sq_l
    Do = Wo.shape[1]

    def body(x_ref, wq_ref, k_ref, v_ref, wo_ref, out_ref,
             kbuf, vbuf, comm, send_sems, recv_sems):
        my = lax.axis_index("i")
        left = lax.rem(my + N_DEV - 1, N_DEV)
        right = lax.rem(my + 1, N_DEV)

        barrier_sem = pltpu.get_barrier_semaphore()
        for nbr in (left, right):
            pl.semaphore_signal(
                barrier_sem, inc=1,
                device_id=(nbr,), device_id_type=pl.DeviceIdType.MESH,
            )
        pl.semaphore_wait(barrier_sem, 2)

        k_loc = k_ref[...].astype(jnp.bfloat16)
        v_loc = v_ref[...].astype(jnp.bfloat16)
        comm[0, 0] = k_loc
        comm[0, 1] = v_loc
        kbuf[:, :, pl.ds(my * Sq_l, Sq_l)] = k_loc.transpose(0, 2, 1, 3)
        vbuf[:, :, pl.ds(my * Sq_l, Sq_l)] = v_loc.transpose(0, 2, 1, 3)

        for h in range(N_DEV - 1):
            rdma = pltpu.make_async_remote_copy(
                src_ref=comm.at[h],
                dst_ref=comm.at[h + 1],
                send_sem=send_sems.at[h],
                recv_sem=recv_sems.at[h],
                device_id=(right,),
                device_id_type=pl.DeviceIdType.MESH,
            )
            rdma.start()
            rdma.wait()
            origin = lax.rem(my + N_DEV - 1 - h, N_DEV)
            kbuf[:, :, pl.ds(origin * Sq_l, Sq_l)] = (
                comm[h + 1, 0].transpose(0, 2, 1, 3))
            vbuf[:, :, pl.ds(origin * Sq_l, Sq_l)] = (
                comm[h + 1, 1].transpose(0, 2, 1, 3))

        wq = wq_ref[...].astype(jnp.bfloat16)
        wo = wo_ref[...].astype(jnp.bfloat16)
        q_start = my * Sq_l
        qb = (q_start + lax.broadcasted_iota(jnp.int32, (Sq_l, S_g), 0)) // BLK
        kb = lax.broadcasted_iota(jnp.int32, (Sq_l, S_g), 1) // BLK
        mask = kb <= qb

        for b in range(B):
            xb = x_ref[b].astype(jnp.bfloat16)
            q = jnp.dot(xb, wq, preferred_element_type=jnp.float32)
            q = q.reshape(Sq_l, Hq, Dh).astype(jnp.bfloat16)
            qh = q.transpose(1, 0, 2)
            kh = kbuf[b]
            scores = lax.dot_general(
                qh, kh, (((2,), (2,)), ((0,), (0,))),
                preferred_element_type=jnp.float32,
            ) * 0.125
            scores = jnp.where(mask[None], scores, -1e9)
            m = jnp.max(scores, axis=-1, keepdims=True)
            w = jnp.exp(scores - m)
            w = w / jnp.sum(w, axis=-1, keepdims=True)
            vh = vbuf[b]
            ctx = lax.dot_general(
                w.astype(jnp.bfloat16), vh, (((2,), (1,)), ((0,), (0,))),
                preferred_element_type=jnp.float32,
            )
            ctx = ctx.transpose(1, 0, 2).reshape(Sq_l, Hq * Dh)
            out_ref[b] = jnp.dot(
                ctx.astype(jnp.bfloat16), wo,
                preferred_element_type=jnp.float32,
            )

    return pl.pallas_call(
        body,
        out_shape=jax.ShapeDtypeStruct((B, Sq_l, Do), jnp.float32),
        in_specs=[pl.BlockSpec(memory_space=pltpu.VMEM)] * 5,
        out_specs=pl.BlockSpec(memory_space=pltpu.VMEM),
        scratch_shapes=[
            pltpu.VMEM((B, Hq, S_g, Dh), jnp.bfloat16),
            pltpu.VMEM((B, Hq, S_g, Dh), jnp.bfloat16),
            pltpu.VMEM((N_DEV, 2, B, Sq_l, Hq, Dh), jnp.bfloat16),
            pltpu.SemaphoreType.DMA((N_DEV - 1,)),
            pltpu.SemaphoreType.DMA((N_DEV - 1,)),
        ],
        compiler_params=pltpu.CompilerParams(collective_id=0),
    )(x, Wq, K_ext, V_ext, Wo)


# device time: 110261 ns/iter; 1.0383x vs baseline; 1.0383x over previous
import jax
import jax.numpy as jnp
from jax import lax
from jax.experimental import pallas as pl
from jax.experimental.pallas import tpu as pltpu

N_DEV = 8
BLK = 64


def kernel(x, Wq, K_ext, V_ext, Wo):
    B, Sq_l, Dm = x.shape
    _, _, Hq, Dh = K_ext.shape
    S_g = N_DEV * Sq_l
    Do = Wo.shape[1]

    def body(x_ref, wq_ref, k_ref, v_ref, wo_ref, out_ref,
             kbuf, vbuf, comm, send_sems, recv_sems):
        my = lax.axis_index("i")
        left = lax.rem(my + N_DEV - 1, N_DEV)
        right = lax.rem(my + 1, N_DEV)

        barrier_sem = pltpu.get_barrier_semaphore()
        for nbr in (left, right):
            pl.semaphore_signal(
                barrier_sem, inc=1,
                device_id=(nbr,), device_id_type=pl.DeviceIdType.MESH,
            )
        pl.semaphore_wait(barrier_sem, 2)

        k_loc = k_ref[...].astype(jnp.bfloat16)
        v_loc = v_ref[...].astype(jnp.bfloat16)
        comm[0, 0] = k_loc
        comm[0, 1] = v_loc
        kbuf[:, :, pl.ds(my * Sq_l, Sq_l)] = k_loc.transpose(0, 2, 1, 3)
        vbuf[:, :, pl.ds(my * Sq_l, Sq_l)] = v_loc.transpose(0, 2, 1, 3)

        def hop(h):
            return pltpu.make_async_remote_copy(
                src_ref=comm.at[h],
                dst_ref=comm.at[h + 1],
                send_sem=send_sems.at[h],
                recv_sem=recv_sems.at[h],
                device_id=(right,),
                device_id_type=pl.DeviceIdType.MESH,
            )

        hop(0).start()
        for h in range(N_DEV - 1):
            hop(h).wait_recv()
            if h + 1 < N_DEV - 1:
                hop(h + 1).start()
            origin = lax.rem(my + N_DEV - 1 - h, N_DEV)
            kbuf[:, :, pl.ds(origin * Sq_l, Sq_l)] = (
                comm[h + 1, 0].transpose(0, 2, 1, 3))
            vbuf[:, :, pl.ds(origin * Sq_l, Sq_l)] = (
                comm[h + 1, 1].transpose(0, 2, 1, 3))

        wq = wq_ref[...].astype(jnp.bfloat16)
        wo = wo_ref[...].astype(jnp.bfloat16)
        q_start = my * Sq_l
        qb = (q_start + lax.broadcasted_iota(jnp.int32, (Sq_l, S_g), 0)) // BLK
        kb = lax.broadcasted_iota(jnp.int32, (Sq_l, S_g), 1) // BLK
        mask = kb <= qb

        for b in range(B):
            xb = x_ref[b].astype(jnp.bfloat16)
            q = jnp.dot(xb, wq, preferred_element_type=jnp.float32)
            q = q.reshape(Sq_l, Hq, Dh).astype(jnp.bfloat16)
            qh = q.transpose(1, 0, 2)
            kh = kbuf[b]
            scores = lax.dot_general(
                qh, kh, (((2,), (2,)), ((0,), (0,))),
                preferred_element_type=jnp.float32,
            ) * 0.125
            scores = jnp.where(mask[None], scores, -1e9)
            m = jnp.max(scores, axis=-1, keepdims=True)
            w = jnp.exp(scores - m)
            w = w / jnp.sum(w, axis=-1, keepdims=True)
            vh = vbuf[b]
            ctx = lax.dot_general(
                w.astype(jnp.bfloat16), vh, (((2,), (1,)), ((0,), (0,))),
                preferred_element_type=jnp.float32,
            )
            ctx = ctx.transpose(1, 0, 2).reshape(Sq_l, Hq * Dh)
            out_ref[b] = jnp.dot(
                ctx.astype(jnp.bfloat16), wo,
                preferred_element_type=jnp.float32,
            )

        for h in range(N_DEV - 1):
            hop(h).wait_send()

    return pl.pallas_call(
        body,
        out_shape=jax.ShapeDtypeStruct((B, Sq_l, Do), jnp.float32),
        in_specs=[pl.BlockSpec(memory_space=pltpu.VMEM)] * 5,
        out_specs=pl.BlockSpec(memory_space=pltpu.VMEM),
        scratch_shapes=[
            pltpu.VMEM((B, Hq, S_g, Dh), jnp.bfloat16),
            pltpu.VMEM((B, Hq, S_g, Dh), jnp.bfloat16),
            pltpu.VMEM((N_DEV, 2, B, Sq_l, Hq, Dh), jnp.bfloat16),
            pltpu.SemaphoreType.DMA((N_DEV - 1,)),
            pltpu.SemaphoreType.DMA((N_DEV - 1,)),
        ],
        compiler_params=pltpu.CompilerParams(collective_id=0),
    )(x, Wq, K_ext, V_ext, Wo)


# device time: 66831 ns/iter; 1.7131x vs baseline; 1.6498x over previous
import jax
import jax.numpy as jnp
from jax import lax
from jax.experimental import pallas as pl
from jax.experimental.pallas import tpu as pltpu

N_DEV = 8
BLK = 64


def kernel(x, Wq, K_ext, V_ext, Wo):
    B, Sq_l, Dm = x.shape
    _, _, Hq, Dh = K_ext.shape
    S_g = N_DEV * Sq_l
    Do = Wo.shape[1]
    HD = Hq * Dh

    def body(x_ref, wq_ref, k_ref, v_ref, wo_ref, out_ref,
             kbuf, vbuf, comm, send_sems, recv_sems):
        my = lax.axis_index("i")
        left = lax.rem(my + N_DEV - 1, N_DEV)
        right = lax.rem(my + 1, N_DEV)

        barrier_sem = pltpu.get_barrier_semaphore()
        for nbr in (left, right):
            pl.semaphore_signal(
                barrier_sem, inc=1,
                device_id=(nbr,), device_id_type=pl.DeviceIdType.MESH,
            )
        pl.semaphore_wait(barrier_sem, 2)

        k_loc = k_ref[...].astype(jnp.bfloat16).reshape(B, Sq_l, HD)
        v_loc = v_ref[...].astype(jnp.bfloat16).reshape(B, Sq_l, HD)
        comm[0, 0] = k_loc
        comm[0, 1] = v_loc
        kbuf[:, pl.ds(my * Sq_l, Sq_l)] = k_loc
        vbuf[:, pl.ds(my * Sq_l, Sq_l)] = v_loc

        def hop(h):
            return pltpu.make_async_remote_copy(
                src_ref=comm.at[h],
                dst_ref=comm.at[h + 1],
                send_sem=send_sems.at[h],
                recv_sem=recv_sems.at[h],
                device_id=(right,),
                device_id_type=pl.DeviceIdType.MESH,
            )

        hop(0).start()
        for h in range(N_DEV - 1):
            hop(h).wait_recv()
            if h + 1 < N_DEV - 1:
                hop(h + 1).start()
            origin = lax.rem(my + N_DEV - 1 - h, N_DEV)
            kbuf[:, pl.ds(origin * Sq_l, Sq_l)] = comm[h + 1, 0]
            vbuf[:, pl.ds(origin * Sq_l, Sq_l)] = comm[h + 1, 1]

        wq = wq_ref[...].astype(jnp.bfloat16)
        wo = wo_ref[...].astype(jnp.bfloat16)
        q_start = my * Sq_l
        qb = (q_start + lax.broadcasted_iota(jnp.int32, (Sq_l, S_g), 0)) // BLK
        kb = lax.broadcasted_iota(jnp.int32, (Sq_l, S_g), 1) // BLK
        bias = jnp.where(kb <= qb, 0.0, -1e9).astype(jnp.float32)

        for b in range(B):
            xb = x_ref[b].astype(jnp.bfloat16)
            q = jnp.dot(xb, wq, preferred_element_type=jnp.float32)
            qbf = q.astype(jnp.bfloat16)
            kf = kbuf[b]
            vf = vbuf[b]
            ctx_cols = []
            for hh in range(Hq):
                sl = slice(hh * Dh, (hh + 1) * Dh)
                s = lax.dot_general(
                    qbf[:, sl], kf[:, sl], (((1,), (1,)), ((), ())),
                    preferred_element_type=jnp.float32,
                )
                w = jnp.exp(s * 0.125 + bias)
                denom = jnp.sum(w, axis=-1, keepdims=True)
                ctx_h = lax.dot_general(
                    w.astype(jnp.bfloat16), vf[:, sl],
                    (((1,), (0,)), ((), ())),
                    preferred_element_type=jnp.float32,
                )
                ctx_cols.append(ctx_h / denom)
            ctx = jnp.concatenate(ctx_cols, axis=1).astype(jnp.bfloat16)
            out_ref[b] = jnp.dot(ctx, wo, preferred_element_type=jnp.float32)

        for h in range(N_DEV - 1):
            hop(h).wait_send()

    return pl.pallas_call(
        body,
        out_shape=jax.ShapeDtypeStruct((B, Sq_l, Do), jnp.float32),
        in_specs=[pl.BlockSpec(memory_space=pltpu.VMEM)] * 5,
        out_specs=pl.BlockSpec(memory_space=pltpu.VMEM),
        scratch_shapes=[
            pltpu.VMEM((B, S_g, HD), jnp.bfloat16),
            pltpu.VMEM((B, S_g, HD), jnp.bfloat16),
            pltpu.VMEM((N_DEV, 2, B, Sq_l, HD), jnp.bfloat16),
            pltpu.SemaphoreType.DMA((N_DEV - 1,)),
            pltpu.SemaphoreType.DMA((N_DEV - 1,)),
        ],
        compiler_params=pltpu.CompilerParams(collective_id=0),
    )(x, Wq, K_ext, V_ext, Wo)


# device time: 44634 ns/iter; 2.5650x vs baseline; 1.4973x over previous
import jax
import jax.numpy as jnp
from jax import lax
from jax.experimental import pallas as pl
from jax.experimental.pallas import tpu as pltpu

N_DEV = 8
BLK = 64
HOPS_R = 4
HOPS_L = 3


def kernel(x, Wq, K_ext, V_ext, Wo):
    B, Sq_l, Dm = x.shape
    _, _, Hq, Dh = K_ext.shape
    S_g = N_DEV * Sq_l
    Do = Wo.shape[1]
    HD = Hq * Dh

    def body(x_ref, wq_ref, k_ref, v_ref, wo_ref, out_ref,
             kbuf, vbuf, comm_r, comm_l,
             send_sems_r, recv_sems_r, send_sems_l, recv_sems_l):
        my = lax.axis_index("i")
        left = lax.rem(my + N_DEV - 1, N_DEV)
        right = lax.rem(my + 1, N_DEV)

        barrier_sem = pltpu.get_barrier_semaphore()
        for nbr in (left, right):
            pl.semaphore_signal(
                barrier_sem, inc=1,
                device_id=(nbr,), device_id_type=pl.DeviceIdType.MESH,
            )
        pl.semaphore_wait(barrier_sem, 2)

        k_loc = k_ref[...].astype(jnp.bfloat16).reshape(B, Sq_l, HD)
        v_loc = v_ref[...].astype(jnp.bfloat16).reshape(B, Sq_l, HD)
        comm_r[0, 0] = k_loc
        comm_r[0, 1] = v_loc
        comm_l[0, 0] = k_loc
        comm_l[0, 1] = v_loc
        kbuf[:, pl.ds(my * Sq_l, Sq_l)] = k_loc
        vbuf[:, pl.ds(my * Sq_l, Sq_l)] = v_loc

        def hop_r(h):
            return pltpu.make_async_remote_copy(
                src_ref=comm_r.at[h],
                dst_ref=comm_r.at[h + 1],
                send_sem=send_sems_r.at[h],
                recv_sem=recv_sems_r.at[h],
                device_id=(right,),
                device_id_type=pl.DeviceIdType.MESH,
            )

        def hop_l(h):
            return pltpu.make_async_remote_copy(
                src_ref=comm_l.at[h],
                dst_ref=comm_l.at[h + 1],
                send_sem=send_sems_l.at[h],
                recv_sem=recv_sems_l.at[h],
                device_id=(left,),
                device_id_type=pl.DeviceIdType.MESH,
            )

        hop_r(0).start()
        hop_l(0).start()
        for h in range(HOPS_R):
            hop_r(h).wait_recv()
            if h + 1 < HOPS_R:
                hop_r(h + 1).start()
            origin = lax.rem(my + N_DEV - 1 - h, N_DEV)
            kbuf[:, pl.ds(origin * Sq_l, Sq_l)] = comm_r[h + 1, 0]
            vbuf[:, pl.ds(origin * Sq_l, Sq_l)] = comm_r[h + 1, 1]
            if h < HOPS_L:
                hop_l(h).wait_recv()
                if h + 1 < HOPS_L:
                    hop_l(h + 1).start()
                origin = lax.rem(my + h + 1, N_DEV)
                kbuf[:, pl.ds(origin * Sq_l, Sq_l)] = comm_l[h + 1, 0]
                vbuf[:, pl.ds(origin * Sq_l, Sq_l)] = comm_l[h + 1, 1]

        wq = wq_ref[...].astype(jnp.bfloat16)
        wo = wo_ref[...].astype(jnp.bfloat16)
        q_start = my * Sq_l
        qb = (q_start + lax.broadcasted_iota(jnp.int32, (Sq_l, S_g), 0)) // BLK
        kb = lax.broadcasted_iota(jnp.int32, (Sq_l, S_g), 1) // BLK
        bias = jnp.where(kb <= qb, 0.0, -1e9).astype(jnp.float32)

        for b in range(B):
            xb = x_ref[b].astype(jnp.bfloat16)
            q = jnp.dot(xb, wq, preferred_element_type=jnp.float32)
            qbf = q.astype(jnp.bfloat16)
            kf = kbuf[b]
            vf = vbuf[b]
            ctx_cols = []
            for hh in range(Hq):
                sl = slice(hh * Dh, (hh + 1) * Dh)
                s = lax.dot_general(
                    qbf[:, sl], kf[:, sl], (((1,), (1,)), ((), ())),
                    preferred_element_type=jnp.float32,
                )
                w = jnp.exp(s * 0.125 + bias)
                denom = jnp.sum(w, axis=-1, keepdims=True)
                ctx_h = lax.dot_general(
                    w.astype(jnp.bfloat16), vf[:, sl],
                    (((1,), (0,)), ((), ())),
                    preferred_element_type=jnp.float32,
                )
                ctx_cols.append(ctx_h / denom)
            ctx = jnp.concatenate(ctx_cols, axis=1).astype(jnp.bfloat16)
            out_ref[b] = jnp.dot(ctx, wo, preferred_element_type=jnp.float32)

        for h in range(HOPS_R):
            hop_r(h).wait_send()
        for h in range(HOPS_L):
            hop_l(h).wait_send()

    return pl.pallas_call(
        body,
        out_shape=jax.ShapeDtypeStruct((B, Sq_l, Do), jnp.float32),
        in_specs=[pl.BlockSpec(memory_space=pltpu.VMEM)] * 5,
        out_specs=pl.BlockSpec(memory_space=pltpu.VMEM),
        scratch_shapes=[
            pltpu.VMEM((B, S_g, HD), jnp.bfloat16),
            pltpu.VMEM((B, S_g, HD), jnp.bfloat16),
            pltpu.VMEM((HOPS_R + 1, 2, B, Sq_l, HD), jnp.bfloat16),
            pltpu.VMEM((HOPS_L + 1, 2, B, Sq_l, HD), jnp.bfloat16),
            pltpu.SemaphoreType.DMA((HOPS_R,)),
            pltpu.SemaphoreType.DMA((HOPS_R,)),
            pltpu.SemaphoreType.DMA((HOPS_L,)),
            pltpu.SemaphoreType.DMA((HOPS_L,)),
        ],
        compiler_params=pltpu.CompilerParams(collective_id=0),
    )(x, Wq, K_ext, V_ext, Wo)


# device time: 39376 ns/iter; 2.9075x vs baseline; 1.1335x over previous
import jax
import jax.numpy as jnp
from jax import lax
from jax.experimental import pallas as pl
from jax.experimental.pallas import tpu as pltpu

N_DEV = 8
BLK = 64
HOPS_R = 4
HOPS_L = 3


def kernel(x, Wq, K_ext, V_ext, Wo):
    B, Sq_l, Dm = x.shape
    _, _, Hq, Dh = K_ext.shape
    Do = Wo.shape[1]
    HD = Hq * Dh

    def body(x_ref, wq_ref, k_ref, v_ref, wo_ref, out_ref,
             comm_r, comm_l,
             send_sems_r, recv_sems_r, send_sems_l, recv_sems_l):
        my = lax.axis_index("i")
        left = lax.rem(my + N_DEV - 1, N_DEV)
        right = lax.rem(my + 1, N_DEV)

        barrier_sem = pltpu.get_barrier_semaphore()
        for nbr in (left, right):
            pl.semaphore_signal(
                barrier_sem, inc=1,
                device_id=(nbr,), device_id_type=pl.DeviceIdType.MESH,
            )
        pl.semaphore_wait(barrier_sem, 2)

        k_loc = k_ref[...].astype(jnp.bfloat16).reshape(B, Sq_l, HD)
        v_loc = v_ref[...].astype(jnp.bfloat16).reshape(B, Sq_l, HD)
        comm_r[0, 0] = k_loc
        comm_r[0, 1] = v_loc
        comm_l[0, 0] = k_loc
        comm_l[0, 1] = v_loc

        def hop_r(h):
            return pltpu.make_async_remote_copy(
                src_ref=comm_r.at[h],
                dst_ref=comm_r.at[h + 1],
                send_sem=send_sems_r.at[h],
                recv_sem=recv_sems_r.at[h],
                device_id=(right,),
                device_id_type=pl.DeviceIdType.MESH,
            )

        def hop_l(h):
            return pltpu.make_async_remote_copy(
                src_ref=comm_l.at[h],
                dst_ref=comm_l.at[h + 1],
                send_sem=send_sems_l.at[h],
                recv_sem=recv_sems_l.at[h],
                device_id=(left,),
                device_id_type=pl.DeviceIdType.MESH,
            )

        hop_r(0).start()
        hop_l(0).start()

        wq = wq_ref[...].astype(jnp.bfloat16)
        wo = wo_ref[...].astype(jnp.bfloat16)
        qbf = [
            jnp.dot(x_ref[b].astype(jnp.bfloat16), wq,
                    preferred_element_type=jnp.float32).astype(jnp.bfloat16)
            for b in range(B)
        ]

        def attend(kc, vc, b, hh, bias=None, vis=None):
            sl = slice(hh * Dh, (hh + 1) * Dh)
            s = lax.dot_general(
                qbf[b][:, sl], kc[:, sl], (((1,), (1,)), ((), ())),
                preferred_element_type=jnp.float32,
            ) * 0.125
            w = jnp.exp(s + bias) if bias is not None else jnp.exp(s) * vis
            den = jnp.sum(w, axis=-1, keepdims=True)
            ctx = lax.dot_general(
                w.astype(jnp.bfloat16), vc[:, sl], (((1,), (0,)), ((), ())),
                preferred_element_type=jnp.float32,
            )
            return ctx, den

        qb = lax.broadcasted_iota(jnp.int32, (Sq_l, Sq_l), 0) // BLK
        kb = lax.broadcasted_iota(jnp.int32, (Sq_l, Sq_l), 1) // BLK
        bias_diag = jnp.where(kb <= qb, 0.0, -1e9).astype(jnp.float32)

        ctx_acc = [[None] * Hq for _ in range(B)]
        den_acc = [[None] * Hq for _ in range(B)]
        for b in range(B):
            for hh in range(Hq):
                ctx_acc[b][hh], den_acc[b][hh] = attend(
                    k_loc[b], v_loc[b], b, hh, bias=bias_diag)

        def consume(comm, h, origin):
            vis = (origin < my).astype(jnp.float32)
            for b in range(B):
                kc = comm[h + 1, 0, b]
                vc = comm[h + 1, 1, b]
                for hh in range(Hq):
                    ctx, den = attend(kc, vc, b, hh, vis=vis)
                    ctx_acc[b][hh] = ctx_acc[b][hh] + ctx
                    den_acc[b][hh] = den_acc[b][hh] + den

        for h in range(HOPS_R):
            hop_r(h).wait_recv()
            if h + 1 < HOPS_R:
                hop_r(h + 1).start()
            consume(comm_r, h, lax.rem(my + N_DEV - 1 - h, N_DEV))
            if h < HOPS_L:
                hop_l(h).wait_recv()
                if h + 1 < HOPS_L:
                    hop_l(h + 1).start()
                consume(comm_l, h, lax.rem(my + h + 1, N_DEV))

        for b in range(B):
            ctx = jnp.concatenate(
                [ctx_acc[b][hh] / den_acc[b][hh] for hh in range(Hq)], axis=1
            ).astype(jnp.bfloat16)
            out_ref[b] = jnp.dot(ctx, wo, preferred_element_type=jnp.float32)

        for h in range(HOPS_R):
            hop_r(h).wait_send()
        for h in range(HOPS_L):
            hop_l(h).wait_send()

    return pl.pallas_call(
        body,
        out_shape=jax.ShapeDtypeStruct((B, Sq_l, Do), jnp.float32),
        in_specs=[pl.BlockSpec(memory_space=pltpu.VMEM)] * 5,
        out_specs=pl.BlockSpec(memory_space=pltpu.VMEM),
        scratch_shapes=[
            pltpu.VMEM((HOPS_R + 1, 2, B, Sq_l, HD), jnp.bfloat16),
            pltpu.VMEM((HOPS_L + 1, 2, B, Sq_l, HD), jnp.bfloat16),
            pltpu.SemaphoreType.DMA((HOPS_R,)),
            pltpu.SemaphoreType.DMA((HOPS_R,)),
            pltpu.SemaphoreType.DMA((HOPS_L,)),
            pltpu.SemaphoreType.DMA((HOPS_L,)),
        ],
        compiler_params=pltpu.CompilerParams(collective_id=0),
    )(x, Wq, K_ext, V_ext, Wo)


# device time: 36310 ns/iter; 3.1530x vs baseline; 1.0844x over previous
import jax
import jax.numpy as jnp
from jax import lax
from jax.experimental import pallas as pl
from jax.experimental.pallas import tpu as pltpu

N_DEV = 8
BLK = 64
HOPS_R = 4
HOPS_L = 3


def kernel(x, Wq, K_ext, V_ext, Wo):
    B, Sq_l, Dm = x.shape
    _, _, Hq, Dh = K_ext.shape
    Do = Wo.shape[1]
    HD = Hq * Dh

    def body(x_ref, wq_ref, k_ref, v_ref, wo_ref, out_ref,
             comm_r, comm_l,
             send_sems_r, recv_sems_r, send_sems_l, recv_sems_l):
        my = lax.axis_index("i")
        left = lax.rem(my + N_DEV - 1, N_DEV)
        right = lax.rem(my + 1, N_DEV)

        barrier_sem = pltpu.get_barrier_semaphore()
        for nbr in (left, right):
            pl.semaphore_signal(
                barrier_sem, inc=1,
                device_id=(nbr,), device_id_type=pl.DeviceIdType.MESH,
            )
        pl.semaphore_wait(barrier_sem, 2)

        k_loc = k_ref[...].astype(jnp.bfloat16).reshape(B, Sq_l, HD)
        v_loc = v_ref[...].astype(jnp.bfloat16).reshape(B, Sq_l, HD)
        comm_r[0, 0] = k_loc
        comm_r[0, 1] = v_loc
        comm_l[0, 0] = k_loc
        comm_l[0, 1] = v_loc

        def hop_r(h, p):
            return pltpu.make_async_remote_copy(
                src_ref=comm_r.at[h, p],
                dst_ref=comm_r.at[h + 1, p],
                send_sem=send_sems_r.at[h, p],
                recv_sem=recv_sems_r.at[h, p],
                device_id=(right,),
                device_id_type=pl.DeviceIdType.MESH,
            )

        def hop_l(h, p):
            return pltpu.make_async_remote_copy(
                src_ref=comm_l.at[h, p],
                dst_ref=comm_l.at[h + 1, p],
                send_sem=send_sems_l.at[h, p],
                recv_sem=recv_sems_l.at[h, p],
                device_id=(left,),
                device_id_type=pl.DeviceIdType.MESH,
            )

        for p in (0, 1):
            hop_r(0, p).start()
            hop_l(0, p).start()

        wq = wq_ref[...].astype(jnp.bfloat16)
        wo = wo_ref[...].astype(jnp.bfloat16)
        qbf = [
            jnp.dot(x_ref[b].astype(jnp.bfloat16), wq,
                    preferred_element_type=jnp.float32).astype(jnp.bfloat16)
            for b in range(B)
        ]

        def attend(kc, vc, b, hh, bias=None, vis=None):
            sl = slice(hh * Dh, (hh + 1) * Dh)
            s = lax.dot_general(
                qbf[b][:, sl], kc[:, sl], (((1,), (1,)), ((), ())),
                preferred_element_type=jnp.float32,
            ) * 0.125
            w = jnp.exp(s + bias) if bias is not None else jnp.exp(s) * vis
            den = jnp.sum(w, axis=-1, keepdims=True)
            ctx = lax.dot_general(
                w.astype(jnp.bfloat16), vc[:, sl], (((1,), (0,)), ((), ())),
                preferred_element_type=jnp.float32,
            )
            return ctx, den

        qb = lax.broadcasted_iota(jnp.int32, (Sq_l, Sq_l), 0) // BLK
        kb = lax.broadcasted_iota(jnp.int32, (Sq_l, Sq_l), 1) // BLK
        bias_diag = jnp.where(kb <= qb, 0.0, -1e9).astype(jnp.float32)

        ctx_acc = [[None] * Hq for _ in range(B)]
        den_acc = [[None] * Hq for _ in range(B)]
        for b in range(B):
            for hh in range(Hq):
                ctx_acc[b][hh], den_acc[b][hh] = attend(
                    k_loc[b], v_loc[b], b, hh, bias=bias_diag)

        def consume(comm, h, origin):
            vis = (origin < my).astype(jnp.float32)
            for b in range(B):
                kc = comm[h + 1, 0, b]
                vc = comm[h + 1, 1, b]
                for hh in range(Hq):
                    ctx, den = attend(kc, vc, b, hh, vis=vis)
                    ctx_acc[b][hh] = ctx_acc[b][hh] + ctx
                    den_acc[b][hh] = den_acc[b][hh] + den

        for h in range(HOPS_R):
            for p in (0, 1):
                hop_r(h, p).wait_recv()
                if h + 1 < HOPS_R:
                    hop_r(h + 1, p).start()
            consume(comm_r, h, lax.rem(my + N_DEV - 1 - h, N_DEV))
            if h < HOPS_L:
                for p in (0, 1):
                    hop_l(h, p).wait_recv()
                    if h + 1 < HOPS_L:
                        hop_l(h + 1, p).start()
                consume(comm_l, h, lax.rem(my + h + 1, N_DEV))

        for b in range(B):
            ctx = jnp.concatenate(
                [ctx_acc[b][hh] / den_acc[b][hh] for hh in range(Hq)], axis=1
            ).astype(jnp.bfloat16)
            out_ref[b] = jnp.dot(ctx, wo, preferred_element_type=jnp.float32)

        for h in range(HOPS_R):
            for p in (0, 1):
                hop_r(h, p).wait_send()
        for h in range(HOPS_L):
            for p in (0, 1):
                hop_l(h, p).wait_send()

    return pl.pallas_call(
        body,
        out_shape=jax.ShapeDtypeStruct((B, Sq_l, Do), jnp.float32),
        in_specs=[pl.BlockSpec(memory_space=pltpu.VMEM)] * 5,
        out_specs=pl.BlockSpec(memory_space=pltpu.VMEM),
        scratch_shapes=[
            pltpu.VMEM((HOPS_R + 1, 2, B, Sq_l, HD), jnp.bfloat16),
            pltpu.VMEM((HOPS_L + 1, 2, B, Sq_l, HD), jnp.bfloat16),
            pltpu.SemaphoreType.DMA((HOPS_R, 2)),
            pltpu.SemaphoreType.DMA((HOPS_R, 2)),
            pltpu.SemaphoreType.DMA((HOPS_L, 2)),
            pltpu.SemaphoreType.DMA((HOPS_L, 2)),
        ],
        compiler_params=pltpu.CompilerParams(collective_id=0),
    )(x, Wq, K_ext, V_ext, Wo)


# device time: 34806 ns/iter; 3.2892x vs baseline; 1.0432x over previous
import jax
import jax.numpy as jnp
from jax import lax
from jax.experimental import pallas as pl
from jax.experimental.pallas import tpu as pltpu

N_DEV = 8
BLK = 64
HOPS_R = 4
HOPS_L = 3


def kernel(x, Wq, K_ext, V_ext, Wo):
    B, Sq_l, Dm = x.shape
    _, _, Hq, Dh = K_ext.shape
    Do = Wo.shape[1]
    HD = Hq * Dh

    def body(x_ref, wq_ref, k_ref, v_ref, wo_ref, out_ref,
             comm_r, comm_l,
             send_sems_r, recv_sems_r, send_sems_l, recv_sems_l):
        my = lax.axis_index("i")
        left = lax.rem(my + N_DEV - 1, N_DEV)
        right = lax.rem(my + 1, N_DEV)

        barrier_sem = pltpu.get_barrier_semaphore()
        for nbr in (left, right):
            pl.semaphore_signal(
                barrier_sem, inc=1,
                device_id=(nbr,), device_id_type=pl.DeviceIdType.MESH,
            )
        pl.semaphore_wait(barrier_sem, 2)

        k_loc = k_ref[...].astype(jnp.bfloat16).reshape(B, Sq_l, HD)
        v_loc = v_ref[...].astype(jnp.bfloat16).reshape(B, Sq_l, HD)
        comm_r[0, 0] = k_loc
        comm_r[0, 1] = v_loc
        comm_l[0, 0] = k_loc
        comm_l[0, 1] = v_loc

        PIECES = [(t, b) for b in range(B) for t in (0, 1)]

        def hop_r(h, t, b):
            return pltpu.make_async_remote_copy(
                src_ref=comm_r.at[h, t, b],
                dst_ref=comm_r.at[h + 1, t, b],
                send_sem=send_sems_r.at[h, 2 * b + t],
                recv_sem=recv_sems_r.at[h, 2 * b + t],
                device_id=(right,),
                device_id_type=pl.DeviceIdType.MESH,
            )

        def hop_l(h, t, b):
            return pltpu.make_async_remote_copy(
                src_ref=comm_l.at[h, t, b],
                dst_ref=comm_l.at[h + 1, t, b],
                send_sem=send_sems_l.at[h, 2 * b + t],
                recv_sem=recv_sems_l.at[h, 2 * b + t],
                device_id=(left,),
                device_id_type=pl.DeviceIdType.MESH,
            )

        for t, b in PIECES:
            hop_r(0, t, b).start()
            hop_l(0, t, b).start()

        wq = wq_ref[...].astype(jnp.bfloat16)
        wo = wo_ref[...].astype(jnp.bfloat16)
        qbf = [
            jnp.dot(x_ref[b].astype(jnp.bfloat16), wq,
                    preferred_element_type=jnp.float32).astype(jnp.bfloat16)
            for b in range(B)
        ]

        def attend(kc, vc, b, hh, bias=None, vis=None):
            sl = slice(hh * Dh, (hh + 1) * Dh)
            s = lax.dot_general(
                qbf[b][:, sl], kc[:, sl], (((1,), (1,)), ((), ())),
                preferred_element_type=jnp.float32,
            ) * 0.125
            w = jnp.exp(s + bias) if bias is not None else jnp.exp(s) * vis
            den = jnp.sum(w, axis=-1, keepdims=True)
            ctx = lax.dot_general(
                w.astype(jnp.bfloat16), vc[:, sl], (((1,), (0,)), ((), ())),
                preferred_element_type=jnp.float32,
            )
            return ctx, den

        qb = lax.broadcasted_iota(jnp.int32, (Sq_l, Sq_l), 0) // BLK
        kb = lax.broadcasted_iota(jnp.int32, (Sq_l, Sq_l), 1) // BLK
        bias_diag = jnp.where(kb <= qb, 0.0, -1e9).astype(jnp.float32)

        ctx_acc = [[None] * Hq for _ in range(B)]
        den_acc = [[None] * Hq for _ in range(B)]
        for b in range(B):
            for hh in range(Hq):
                ctx_acc[b][hh], den_acc[b][hh] = attend(
                    k_loc[b], v_loc[b], b, hh, bias=bias_diag)

        def consume(comm, h, origin, b):
            vis = (origin < my).astype(jnp.float32)
            kc = comm[h + 1, 0, b]
            vc = comm[h + 1, 1, b]
            for hh in range(Hq):
                ctx, den = attend(kc, vc, b, hh, vis=vis)
                ctx_acc[b][hh] = ctx_acc[b][hh] + ctx
                den_acc[b][hh] = den_acc[b][hh] + den

        for h in range(HOPS_R):
            for b in range(B):
                for t in (0, 1):
                    hop_r(h, t, b).wait_recv()
                    if h + 1 < HOPS_R:
                        hop_r(h + 1, t, b).start()
                consume(comm_r, h, lax.rem(my + N_DEV - 1 - h, N_DEV), b)
            if h < HOPS_L:
                for b in range(B):
                    for t in (0, 1):
                        hop_l(h, t, b).wait_recv()
                        if h + 1 < HOPS_L:
                            hop_l(h + 1, t, b).start()
                    consume(comm_l, h, lax.rem(my + h + 1, N_DEV), b)

        for b in range(B):
            ctx = jnp.concatenate(
                [ctx_acc[b][hh] / den_acc[b][hh] for hh in range(Hq)], axis=1
            ).astype(jnp.bfloat16)
            out_ref[b] = jnp.dot(ctx, wo, preferred_element_type=jnp.float32)

        for h in range(HOPS_R):
            for t, b in PIECES:
                hop_r(h, t, b).wait_send()
        for h in range(HOPS_L):
            for t, b in PIECES:
                hop_l(h, t, b).wait_send()

    return pl.pallas_call(
        body,
        out_shape=jax.ShapeDtypeStruct((B, Sq_l, Do), jnp.float32),
        in_specs=[pl.BlockSpec(memory_space=pltpu.VMEM)] * 5,
        out_specs=pl.BlockSpec(memory_space=pltpu.VMEM),
        scratch_shapes=[
            pltpu.VMEM((HOPS_R + 1, 2, B, Sq_l, HD), jnp.bfloat16),
            pltpu.VMEM((HOPS_L + 1, 2, B, Sq_l, HD), jnp.bfloat16),
            pltpu.SemaphoreType.DMA((HOPS_R, 4)),
            pltpu.SemaphoreType.DMA((HOPS_R, 4)),
            pltpu.SemaphoreType.DMA((HOPS_L, 4)),
            pltpu.SemaphoreType.DMA((HOPS_L, 4)),
        ],
        compiler_params=pltpu.CompilerParams(collective_id=0),
    )(x, Wq, K_ext, V_ext, Wo)


# device time: 27325 ns/iter; 4.1898x vs baseline; 1.2738x over previous
import jax
import jax.numpy as jnp
from jax import lax
from jax.experimental import pallas as pl
from jax.experimental.pallas import tpu as pltpu

N_DEV = 8
BLK = 64
HOPS_R = 4
HOPS_L = 3


def kernel(x, Wq, K_ext, V_ext, Wo):
    B, Sq_l, Dm = x.shape
    _, _, Hq, Dh = K_ext.shape
    Do = Wo.shape[1]
    HD = Hq * Dh

    def body(x_ref, wq_ref, k_ref, v_ref, wo_ref, out_ref,
             comm_r, comm_l, scomm_r, scomm_l,
             send_sems_r, recv_sems_r, send_sems_l, recv_sems_l):
        my = lax.axis_index("i")
        left = lax.rem(my + N_DEV - 1, N_DEV)
        right = lax.rem(my + 1, N_DEV)

        barrier_sem = pltpu.get_barrier_semaphore()
        for nbr in (left, right):
            pl.semaphore_signal(
                barrier_sem, inc=1,
                device_id=(nbr,), device_id_type=pl.DeviceIdType.MESH,
            )
        pl.semaphore_wait(barrier_sem, 2)

        k_loc = k_ref[...].astype(jnp.bfloat16).reshape(B, Sq_l, HD)
        v_loc = v_ref[...].astype(jnp.bfloat16).reshape(B, Sq_l, HD)

        for b in range(B):
            for t, arr in ((0, k_loc[b]), (1, v_loc[b])):
                af = arr.astype(jnp.float32)
                head_scales = []
                for hh in range(Hq):
                    sl = slice(hh * Dh, (hh + 1) * Dh)
                    m = jnp.maximum(jnp.max(jnp.abs(af[:, sl])), 1e-6)
                    scale = m / 127.0
                    head_scales.append(scale.reshape(1, 1))
                    q = jnp.clip(
                        jnp.round(af[:, sl] / scale), -127.0, 127.0
                    ).astype(jnp.int8)
                    comm_r[0, t, b, :, sl] = q
                    comm_l[0, t, b, :, sl] = q
                scale_row = jnp.concatenate(head_scales, axis=1)
                scomm_r[0, t, b] = scale_row
                scomm_l[0, t, b] = scale_row

        PIECES = [(t, b) for b in range(B) for t in (0, 1)]

        def hop_r(h, t, b):
            return pltpu.make_async_remote_copy(
                src_ref=comm_r.at[h, t, b],
                dst_ref=comm_r.at[h + 1, t, b],
                send_sem=send_sems_r.at[h, 2 * b + t],
                recv_sem=recv_sems_r.at[h, 2 * b + t],
                device_id=(right,),
                device_id_type=pl.DeviceIdType.MESH,
            )

        def hop_l(h, t, b):
            return pltpu.make_async_remote_copy(
                src_ref=comm_l.at[h, t, b],
                dst_ref=comm_l.at[h + 1, t, b],
                send_sem=send_sems_l.at[h, 2 * b + t],
                recv_sem=recv_sems_l.at[h, 2 * b + t],
                device_id=(left,),
                device_id_type=pl.DeviceIdType.MESH,
            )

        def hop_rs(h):
            return pltpu.make_async_remote_copy(
                src_ref=scomm_r.at[h],
                dst_ref=scomm_r.at[h + 1],
                send_sem=send_sems_r.at[h, 4],
                recv_sem=recv_sems_r.at[h, 4],
                device_id=(right,),
                device_id_type=pl.DeviceIdType.MESH,
            )

        def hop_ls(h):
            return pltpu.make_async_remote_copy(
                src_ref=scomm_l.at[h],
                dst_ref=scomm_l.at[h + 1],
                send_sem=send_sems_l.at[h, 4],
                recv_sem=recv_sems_l.at[h, 4],
                device_id=(left,),
                device_id_type=pl.DeviceIdType.MESH,
            )

        hop_rs(0).start()
        hop_ls(0).start()
        for t, b in PIECES:
            hop_r(0, t, b).start()
            hop_l(0, t, b).start()

        wq = wq_ref[...].astype(jnp.bfloat16)
        wo = wo_ref[...].astype(jnp.bfloat16)
        qbf = [
            jnp.dot(x_ref[b].astype(jnp.bfloat16), wq,
                    preferred_element_type=jnp.float32).astype(jnp.bfloat16)
            for b in range(B)
        ]

        def attend(kc, vc, b, hh, bias=None, vis=None,
                   k_scale=None, v_scale=None):
            sl = slice(hh * Dh, (hh + 1) * Dh)
            smul = 0.125 if k_scale is None else 0.125 * k_scale
            s = lax.dot_general(
                qbf[b][:, sl], kc[:, sl], (((1,), (1,)), ((), ())),
                preferred_element_type=jnp.float32,
            ) * smul
            w = jnp.exp(s + bias) if bias is not None else jnp.exp(s)
            den = jnp.sum(w, axis=-1, keepdims=True)
            ctx = lax.dot_general(
                w.astype(jnp.bfloat16), vc[:, sl], (((1,), (0,)), ((), ())),
                preferred_element_type=jnp.float32,
            )
            cmul = vis if v_scale is None else (
                v_scale if vis is None else vis * v_scale)
            if cmul is not None:
                ctx = ctx * cmul
                den = den * (vis if vis is not None else 1.0)
            return ctx, den

        qb = lax.broadcasted_iota(jnp.int32, (Sq_l, Sq_l), 0) // BLK
        kb = lax.broadcasted_iota(jnp.int32, (Sq_l, Sq_l), 1) // BLK
        bias_diag = jnp.where(kb <= qb, 0.0, -1e9).astype(jnp.float32)

        ctx_acc = [[None] * Hq for _ in range(B)]
        den_acc = [[None] * Hq for _ in range(B)]
        for b in range(B):
            for hh in range(Hq):
                ctx_acc[b][hh], den_acc[b][hh] = attend(
                    k_loc[b], v_loc[b], b, hh, bias=bias_diag)

        def consume(comm, scs, h, origin, b):
            vis = (origin < my).astype(jnp.float32)
            kc = comm[h + 1, 0, b].astype(jnp.bfloat16)
            vc = comm[h + 1, 1, b].astype(jnp.bfloat16)
            for hh in range(Hq):
                ctx, den = attend(
                    kc, vc, b, hh, vis=vis,
                    k_scale=scs[0, b, 0, hh], v_scale=scs[1, b, 0, hh])
                ctx_acc[b][hh] = ctx_acc[b][hh] + ctx
                den_acc[b][hh] = den_acc[b][hh] + den

        for h in range(HOPS_R):
            hop_rs(h).wait_recv()
            if h + 1 < HOPS_R:
                hop_rs(h + 1).start()
            scs_r = scomm_r[h + 1]
            for b in range(B):
                for t in (0, 1):
                    hop_r(h, t, b).wait_recv()
                    if h + 1 < HOPS_R:
                        hop_r(h + 1, t, b).start()
                consume(comm_r, scs_r, h, lax.rem(my + N_DEV - 1 - h, N_DEV), b)
            if h < HOPS_L:
                hop_ls(h).wait_recv()
                if h + 1 < HOPS_L:
                    hop_ls(h + 1).start()
                scs_l = scomm_l[h + 1]
                for b in range(B):
                    for t in (0, 1):
                        hop_l(h, t, b).wait_recv()
                        if h + 1 < HOPS_L:
                            hop_l(h + 1, t, b).start()
                    consume(comm_l, scs_l, h, lax.rem(my + h + 1, N_DEV), b)

        for b in range(B):
            ctx = jnp.concatenate(
                [ctx_acc[b][hh] / den_acc[b][hh] for hh in range(Hq)], axis=1
            ).astype(jnp.bfloat16)
            out_ref[b] = jnp.dot(ctx, wo, preferred_element_type=jnp.float32)

        for h in range(HOPS_R):
            hop_rs(h).wait_send()
            for t, b in PIECES:
                hop_r(h, t, b).wait_send()
        for h in range(HOPS_L):
            hop_ls(h).wait_send()
            for t, b in PIECES:
                hop_l(h, t, b).wait_send()

    return pl.pallas_call(
        body,
        out_shape=jax.ShapeDtypeStruct((B, Sq_l, Do), jnp.float32),
        in_specs=[pl.BlockSpec(memory_space=pltpu.VMEM)] * 5,
        out_specs=pl.BlockSpec(memory_space=pltpu.VMEM),
        scratch_shapes=[
            pltpu.VMEM((HOPS_R + 1, 2, B, Sq_l, HD), jnp.int8),
            pltpu.VMEM((HOPS_L + 1, 2, B, Sq_l, HD), jnp.int8),
            pltpu.VMEM((HOPS_R + 1, 2, B, 1, Hq), jnp.float32),
            pltpu.VMEM((HOPS_L + 1, 2, B, 1, Hq), jnp.float32),
            pltpu.SemaphoreType.DMA((HOPS_R, 5)),
            pltpu.SemaphoreType.DMA((HOPS_R, 5)),
            pltpu.SemaphoreType.DMA((HOPS_L, 5)),
            pltpu.SemaphoreType.DMA((HOPS_L, 5)),
        ],
        compiler_params=pltpu.CompilerParams(collective_id=0),
    )(x, Wq, K_ext, V_ext, Wo)


# device time: 25882 ns/iter; 4.4233x vs baseline; 1.0558x over previous
import jax
import jax.numpy as jnp
from jax import lax
from jax.experimental import pallas as pl
from jax.experimental.pallas import tpu as pltpu

N_DEV = 8
BLK = 64
HOPS_R = 3
HOPS_L = 3


def kernel(x, Wq, K_ext, V_ext, Wo):
    B, Sq_l, Dm = x.shape
    _, _, Hq, Dh = K_ext.shape
    Do = Wo.shape[1]
    HD = Hq * Dh

    def body(x_ref, wq_ref, k_ref, v_ref, wo_ref, out_ref,
             comm_r, comm_l, comm_z, scomm_r, scomm_l, scomm_z,
             send_sems_r, recv_sems_r, send_sems_l, recv_sems_l,
             send_sems_z, recv_sems_z):
        my = lax.axis_index("i")
        left = lax.rem(my + N_DEV - 1, N_DEV)
        right = lax.rem(my + 1, N_DEV)
        zp = lax.rem(my + 4, N_DEV)

        barrier_sem = pltpu.get_barrier_semaphore()
        for nbr in (left, right, zp):
            pl.semaphore_signal(
                barrier_sem, inc=1,
                device_id=(nbr,), device_id_type=pl.DeviceIdType.MESH,
            )
        pl.semaphore_wait(barrier_sem, 3)

        k_loc = k_ref[...].astype(jnp.bfloat16).reshape(B, Sq_l, HD)
        v_loc = v_ref[...].astype(jnp.bfloat16).reshape(B, Sq_l, HD)

        for b in range(B):
            for t, arr in ((0, k_loc[b]), (1, v_loc[b])):
                af = arr.astype(jnp.float32)
                head_scales = []
                for hh in range(Hq):
                    sl = slice(hh * Dh, (hh + 1) * Dh)
                    m = jnp.maximum(jnp.max(jnp.abs(af[:, sl])), 1e-6)
                    scale = m / 127.0
                    head_scales.append(scale.reshape(1, 1))
                    q = jnp.clip(
                        jnp.round(af[:, sl] / scale), -127.0, 127.0
                    ).astype(jnp.int8)
                    comm_r[0, t, b, :, sl] = q
                    comm_l[0, t, b, :, sl] = q
                scale_row = jnp.concatenate(head_scales, axis=1)
                scomm_r[0, t, b] = scale_row
                scomm_l[0, t, b] = scale_row

        PIECES = [(t, b) for b in range(B) for t in (0, 1)]

        def hop_r(h, t, b):
            return pltpu.make_async_remote_copy(
                src_ref=comm_r.at[h, t, b],
                dst_ref=comm_r.at[h + 1, t, b],
                send_sem=send_sems_r.at[h, 2 * b + t],
                recv_sem=recv_sems_r.at[h, 2 * b + t],
                device_id=(right,),
                device_id_type=pl.DeviceIdType.MESH,
            )

        def hop_l(h, t, b):
            return pltpu.make_async_remote_copy(
                src_ref=comm_l.at[h, t, b],
                dst_ref=comm_l.at[h + 1, t, b],
                send_sem=send_sems_l.at[h, 2 * b + t],
                recv_sem=recv_sems_l.at[h, 2 * b + t],
                device_id=(left,),
                device_id_type=pl.DeviceIdType.MESH,
            )

        def hop_rs(h):
            return pltpu.make_async_remote_copy(
                src_ref=scomm_r.at[h],
                dst_ref=scomm_r.at[h + 1],
                send_sem=send_sems_r.at[h, 4],
                recv_sem=recv_sems_r.at[h, 4],
                device_id=(right,),
                device_id_type=pl.DeviceIdType.MESH,
            )

        def hop_ls(h):
            return pltpu.make_async_remote_copy(
                src_ref=scomm_l.at[h],
                dst_ref=scomm_l.at[h + 1],
                send_sem=send_sems_l.at[h, 4],
                recv_sem=recv_sems_l.at[h, 4],
                device_id=(left,),
                device_id_type=pl.DeviceIdType.MESH,
            )

        def push_z(t, b):
            return pltpu.make_async_remote_copy(
                src_ref=comm_r.at[0, t, b],
                dst_ref=comm_z.at[t, b],
                send_sem=send_sems_z.at[2 * b + t],
                recv_sem=recv_sems_z.at[2 * b + t],
                device_id=(zp,),
                device_id_type=pl.DeviceIdType.MESH,
            )

        def push_zs():
            return pltpu.make_async_remote_copy(
                src_ref=scomm_r.at[0],
                dst_ref=scomm_z,
                send_sem=send_sems_z.at[4],
                recv_sem=recv_sems_z.at[4],
                device_id=(zp,),
                device_id_type=pl.DeviceIdType.MESH,
            )

        hop_rs(0).start()
        hop_ls(0).start()
        push_zs().start()
        for t, b in PIECES:
            hop_r(0, t, b).start()
            hop_l(0, t, b).start()
            push_z(t, b).start()

        wq = wq_ref[...].astype(jnp.bfloat16)
        wo = wo_ref[...].astype(jnp.bfloat16)
        qbf = [
            jnp.dot(x_ref[b].astype(jnp.bfloat16), wq,
                    preferred_element_type=jnp.float32).astype(jnp.bfloat16)
            for b in range(B)
        ]

        def attend(kc, vc, b, hh, bias=None, vis=None,
                   k_scale=None, v_scale=None):
            sl = slice(hh * Dh, (hh + 1) * Dh)
            smul = 0.125 if k_scale is None else 0.125 * k_scale
            s = lax.dot_general(
                qbf[b][:, sl], kc[:, sl], (((1,), (1,)), ((), ())),
                preferred_element_type=jnp.float32,
            ) * smul
            w = jnp.exp(s + bias) if bias is not None else jnp.exp(s)
            den = jnp.sum(w, axis=-1, keepdims=True)
            ctx = lax.dot_general(
                w.astype(jnp.bfloat16), vc[:, sl], (((1,), (0,)), ((), ())),
                preferred_element_type=jnp.float32,
            )
            cmul = vis if v_scale is None else (
                v_scale if vis is None else vis * v_scale)
            if cmul is not None:
                ctx = ctx * cmul
                den = den * (vis if vis is not None else 1.0)
            return ctx, den

        qb = lax.broadcasted_iota(jnp.int32, (Sq_l, Sq_l), 0) // BLK
        kb = lax.broadcasted_iota(jnp.int32, (Sq_l, Sq_l), 1) // BLK
        bias_diag = jnp.where(kb <= qb, 0.0, -1e9).astype(jnp.float32)

        ctx_acc = [[None] * Hq for _ in range(B)]
        den_acc = [[None] * Hq for _ in range(B)]
        for b in range(B):
            for hh in range(Hq):
                ctx_acc[b][hh], den_acc[b][hh] = attend(
                    k_loc[b], v_loc[b], b, hh, bias=bias_diag)

        def consume_chunk(kc_i8, vc_i8, scs, origin, b):
            vis = (origin < my).astype(jnp.float32)
            kc = kc_i8.astype(jnp.bfloat16)
            vc = vc_i8.astype(jnp.bfloat16)
            for hh in range(Hq):
                ctx, den = attend(
                    kc, vc, b, hh, vis=vis,
                    k_scale=scs[0, b, 0, hh], v_scale=scs[1, b, 0, hh])
                ctx_acc[b][hh] = ctx_acc[b][hh] + ctx
                den_acc[b][hh] = den_acc[b][hh] + den

        def consume(comm, scs, h, origin, b):
            consume_chunk(comm[h + 1, 0, b], comm[h + 1, 1, b],
                          scs, origin, b)

        for h in range(HOPS_R):
            hop_rs(h).wait_recv()
            if h + 1 < HOPS_R:
                hop_rs(h + 1).start()
            scs_r = scomm_r[h + 1]
            for b in range(B):
                for t in (0, 1):
                    hop_r(h, t, b).wait_recv()
                    if h + 1 < HOPS_R:
                        hop_r(h + 1, t, b).start()
                consume(comm_r, scs_r, h, lax.rem(my + N_DEV - 1 - h, N_DEV), b)
            if h < HOPS_L:
                hop_ls(h).wait_recv()
                if h + 1 < HOPS_L:
                    hop_ls(h + 1).start()
                scs_l = scomm_l[h + 1]
                for b in range(B):
                    for t in (0, 1):
                        hop_l(h, t, b).wait_recv()
                        if h + 1 < HOPS_L:
                            hop_l(h + 1, t, b).start()
                    consume(comm_l, scs_l, h, lax.rem(my + h + 1, N_DEV), b)
            if h == 1:
                push_zs().wait_recv()
                scs_z = scomm_z[...]
                for b in range(B):
                    for t in (0, 1):
                        push_z(t, b).wait_recv()
                    consume_chunk(comm_z[0, b], comm_z[1, b], scs_z, zp, b)

        for b in range(B):
            ctx = jnp.concatenate(
                [ctx_acc[b][hh] / den_acc[b][hh] for hh in range(Hq)], axis=1
            ).astype(jnp.bfloat16)
            out_ref[b] = jnp.dot(ctx, wo, preferred_element_type=jnp.float32)

        for h in range(HOPS_R):
            hop_rs(h).wait_send()
            for t, b in PIECES:
                hop_r(h, t, b).wait_send()
        for h in range(HOPS_L):
            hop_ls(h).wait_send()
            for t, b in PIECES:
                hop_l(h, t, b).wait_send()
        push_zs().wait_send()
        for t, b in PIECES:
            push_z(t, b).wait_send()

    return pl.pallas_call(
        body,
        out_shape=jax.ShapeDtypeStruct((B, Sq_l, Do), jnp.float32),
        in_specs=[pl.BlockSpec(memory_space=pltpu.VMEM)] * 5,
        out_specs=pl.BlockSpec(memory_space=pltpu.VMEM),
        scratch_shapes=[
            pltpu.VMEM((HOPS_R + 1, 2, B, Sq_l, HD), jnp.int8),
            pltpu.VMEM((HOPS_L + 1, 2, B, Sq_l, HD), jnp.int8),
            pltpu.VMEM((2, B, Sq_l, HD), jnp.int8),
            pltpu.VMEM((HOPS_R + 1, 2, B, 1, Hq), jnp.float32),
            pltpu.VMEM((HOPS_L + 1, 2, B, 1, Hq), jnp.float32),
            pltpu.VMEM((2, B, 1, Hq), jnp.float32),
            pltpu.SemaphoreType.DMA((HOPS_R, 5)),
            pltpu.SemaphoreType.DMA((HOPS_R, 5)),
            pltpu.SemaphoreType.DMA((HOPS_L, 5)),
            pltpu.SemaphoreType.DMA((HOPS_L, 5)),
            pltpu.SemaphoreType.DMA((5,)),
            pltpu.SemaphoreType.DMA((5,)),
        ],
        compiler_params=pltpu.CompilerParams(collective_id=0),
    )(x, Wq, K_ext, V_ext, Wo)


# device time: 24954 ns/iter; 4.5878x vs baseline; 1.0372x over previous
import jax
import jax.numpy as jnp
from jax import lax
from jax.experimental import pallas as pl
from jax.experimental.pallas import tpu as pltpu

N_DEV = 8
BLK = 64
HOPS_R = 3
HOPS_L = 3


def kernel(x, Wq, K_ext, V_ext, Wo):
    B, Sq_l, Dm = x.shape
    _, _, Hq, Dh = K_ext.shape
    Do = Wo.shape[1]
    HD = Hq * Dh

    def body(x_ref, wq_ref, k_ref, v_ref, wo_ref, out_ref,
             comm_r, comm_l, comm_z, scomm_r, scomm_l, scomm_z,
             send_sems_r, recv_sems_r, send_sems_l, recv_sems_l,
             send_sems_z, recv_sems_z):
        my = lax.axis_index("i")
        left = lax.rem(my + N_DEV - 1, N_DEV)
        right = lax.rem(my + 1, N_DEV)
        zp = lax.rem(my + 4, N_DEV)

        barrier_sem = pltpu.get_barrier_semaphore()
        for nbr in (left, right, zp):
            pl.semaphore_signal(
                barrier_sem, inc=1,
                device_id=(nbr,), device_id_type=pl.DeviceIdType.MESH,
            )

        k_loc = k_ref[...].astype(jnp.bfloat16).reshape(B, Sq_l, HD)
        v_loc = v_ref[...].astype(jnp.bfloat16).reshape(B, Sq_l, HD)

        afs = {}
        scales = {}
        for t, src in ((0, k_loc), (1, v_loc)):
            for b in range(B):
                af = src[b].astype(jnp.float32)
                afs[t, b] = af
                head_scales = []
                for hh in range(Hq):
                    sl = slice(hh * Dh, (hh + 1) * Dh)
                    m = jnp.maximum(jnp.max(jnp.abs(af[:, sl])), 1e-6)
                    head_scales.append(m / 127.0)
                scales[t, b] = head_scales
                scale_row = jnp.concatenate(
                    [s.reshape(1, 1) for s in head_scales], axis=1)
                scomm_r[0, t, b] = scale_row
                scomm_l[0, t, b] = scale_row

        PIECES = [(t, b) for b in range(B) for t in (0, 1)]

        def hop_r(h, t, b):
            return pltpu.make_async_remote_copy(
                src_ref=comm_r.at[h, t, b],
                dst_ref=comm_r.at[h + 1, t, b],
                send_sem=send_sems_r.at[h, 2 * b + t],
                recv_sem=recv_sems_r.at[h, 2 * b + t],
                device_id=(right,),
                device_id_type=pl.DeviceIdType.MESH,
            )

        def hop_l(h, t, b):
            return pltpu.make_async_remote_copy(
                src_ref=comm_l.at[h, t, b],
                dst_ref=comm_l.at[h + 1, t, b],
                send_sem=send_sems_l.at[h, 2 * b + t],
                recv_sem=recv_sems_l.at[h, 2 * b + t],
                device_id=(left,),
                device_id_type=pl.DeviceIdType.MESH,
            )

        def hop_rs(h):
            return pltpu.make_async_remote_copy(
                src_ref=scomm_r.at[h],
                dst_ref=scomm_r.at[h + 1],
                send_sem=send_sems_r.at[h, 4],
                recv_sem=recv_sems_r.at[h, 4],
                device_id=(right,),
                device_id_type=pl.DeviceIdType.MESH,
            )

        def hop_ls(h):
            return pltpu.make_async_remote_copy(
                src_ref=scomm_l.at[h],
                dst_ref=scomm_l.at[h + 1],
                send_sem=send_sems_l.at[h, 4],
                recv_sem=recv_sems_l.at[h, 4],
                device_id=(left,),
                device_id_type=pl.DeviceIdType.MESH,
            )

        def push_z(t, b):
            return pltpu.make_async_remote_copy(
                src_ref=comm_r.at[0, t, b],
                dst_ref=comm_z.at[t, b],
                send_sem=send_sems_z.at[2 * b + t],
                recv_sem=recv_sems_z.at[2 * b + t],
                device_id=(zp,),
                device_id_type=pl.DeviceIdType.MESH,
            )

        def push_zs():
            return pltpu.make_async_remote_copy(
                src_ref=scomm_r.at[0],
                dst_ref=scomm_z,
                send_sem=send_sems_z.at[4],
                recv_sem=recv_sems_z.at[4],
                device_id=(zp,),
                device_id_type=pl.DeviceIdType.MESH,
            )

        pl.semaphore_wait(barrier_sem, 3)
        hop_rs(0).start()
        hop_ls(0).start()
        push_zs().start()
        for t, b in PIECES:
            for hh in range(Hq):
                sl = slice(hh * Dh, (hh + 1) * Dh)
                q = jnp.clip(
                    jnp.round(afs[t, b][:, sl] / scales[t, b][hh]),
                    -127.0, 127.0,
                ).astype(jnp.int8)
                comm_r[0, t, b, :, sl] = q
                comm_l[0, t, b, :, sl] = q
            hop_r(0, t, b).start()
            hop_l(0, t, b).start()
            push_z(t, b).start()

        wq = wq_ref[...].astype(jnp.bfloat16)
        wo = wo_ref[...].astype(jnp.bfloat16)
        qbf = [
            jnp.dot(x_ref[b].astype(jnp.bfloat16), wq,
                    preferred_element_type=jnp.float32).astype(jnp.bfloat16)
            for b in range(B)
        ]

        def attend(kc, vc, b, hh, bias=None, vis=None,
                   k_scale=None, v_scale=None):
            sl = slice(hh * Dh, (hh + 1) * Dh)
            smul = 0.125 if k_scale is None else 0.125 * k_scale
            s = lax.dot_general(
                qbf[b][:, sl], kc[:, sl], (((1,), (1,)), ((), ())),
                preferred_element_type=jnp.float32,
            ) * smul
            w = jnp.exp(s + bias) if bias is not None else jnp.exp(s)
            den = jnp.sum(w, axis=-1, keepdims=True)
            ctx = lax.dot_general(
                w.astype(jnp.bfloat16), vc[:, sl], (((1,), (0,)), ((), ())),
                preferred_element_type=jnp.float32,
            )
            cmul = vis if v_scale is None else (
                v_scale if vis is None else vis * v_scale)
            if cmul is not None:
                ctx = ctx * cmul
                den = den * (vis if vis is not None else 1.0)
            return ctx, den

        qb = lax.broadcasted_iota(jnp.int32, (Sq_l, Sq_l), 0) // BLK
        kb = lax.broadcasted_iota(jnp.int32, (Sq_l, Sq_l), 1) // BLK
        bias_diag = jnp.where(kb <= qb, 0.0, -1e9).astype(jnp.float32)

        ctx_acc = [[None] * Hq for _ in range(B)]
        den_acc = [[None] * Hq for _ in range(B)]
        for b in range(B):
            for hh in range(Hq):
                ctx_acc[b][hh], den_acc[b][hh] = attend(
                    k_loc[b], v_loc[b], b, hh, bias=bias_diag)

        def consume_chunk(kc_i8, vc_i8, scs, origin, b):
            vis = (origin < my).astype(jnp.float32)
            kc = kc_i8.astype(jnp.bfloat16)
            vc = vc_i8.astype(jnp.bfloat16)
            for hh in range(Hq):
                ctx, den = attend(
                    kc, vc, b, hh, vis=vis,
                    k_scale=scs[0, b, 0, hh], v_scale=scs[1, b, 0, hh])
                ctx_acc[b][hh] = ctx_acc[b][hh] + ctx
                den_acc[b][hh] = den_acc[b][hh] + den

        def consume(comm, scs, h, origin, b):
            consume_chunk(comm[h + 1, 0, b], comm[h + 1, 1, b],
                          scs, origin, b)

        for h in range(HOPS_R):
            hop_rs(h).wait_recv()
            if h + 1 < HOPS_R:
                hop_rs(h + 1).start()
            scs_r = scomm_r[h + 1]
            for b in range(B):
                for t in (0, 1):
                    hop_r(h, t, b).wait_recv()
                    if h + 1 < HOPS_R:
                        hop_r(h + 1, t, b).start()
                consume(comm_r, scs_r, h, lax.rem(my + N_DEV - 1 - h, N_DEV), b)
            if h < HOPS_L:
                hop_ls(h).wait_recv()
                if h + 1 < HOPS_L:
                    hop_ls(h + 1).start()
                scs_l = scomm_l[h + 1]
                for b in range(B):
                    for t in (0, 1):
                        hop_l(h, t, b).wait_recv()
                        if h + 1 < HOPS_L:
                            hop_l(h + 1, t, b).start()
                    consume(comm_l, scs_l, h, lax.rem(my + h + 1, N_DEV), b)
            if h == 1:
                push_zs().wait_recv()
                scs_z = scomm_z[...]
                for b in range(B):
                    for t in (0, 1):
                        push_z(t, b).wait_recv()
                    consume_chunk(comm_z[0, b], comm_z[1, b], scs_z, zp, b)

        for b in range(B):
            ctx = jnp.concatenate(
                [ctx_acc[b][hh] / den_acc[b][hh] for hh in range(Hq)], axis=1
            ).astype(jnp.bfloat16)
            out_ref[b] = jnp.dot(ctx, wo, preferred_element_type=jnp.float32)

        for h in range(HOPS_R):
            hop_rs(h).wait_send()
            for t, b in PIECES:
                hop_r(h, t, b).wait_send()
        for h in range(HOPS_L):
            hop_ls(h).wait_send()
            for t, b in PIECES:
                hop_l(h, t, b).wait_send()
        push_zs().wait_send()
        for t, b in PIECES:
            push_z(t, b).wait_send()

    return pl.pallas_call(
        body,
        out_shape=jax.ShapeDtypeStruct((B, Sq_l, Do), jnp.float32),
        in_specs=[pl.BlockSpec(memory_space=pltpu.VMEM)] * 5,
        out_specs=pl.BlockSpec(memory_space=pltpu.VMEM),
        scratch_shapes=[
            pltpu.VMEM((HOPS_R + 1, 2, B, Sq_l, HD), jnp.int8),
            pltpu.VMEM((HOPS_L + 1, 2, B, Sq_l, HD), jnp.int8),
            pltpu.VMEM((2, B, Sq_l, HD), jnp.int8),
            pltpu.VMEM((HOPS_R + 1, 2, B, 1, Hq), jnp.float32),
            pltpu.VMEM((HOPS_L + 1, 2, B, 1, Hq), jnp.float32),
            pltpu.VMEM((2, B, 1, Hq), jnp.float32),
            pltpu.SemaphoreType.DMA((HOPS_R, 5)),
            pltpu.SemaphoreType.DMA((HOPS_R, 5)),
            pltpu.SemaphoreType.DMA((HOPS_L, 5)),
            pltpu.SemaphoreType.DMA((HOPS_L, 5)),
            pltpu.SemaphoreType.DMA((5,)),
            pltpu.SemaphoreType.DMA((5,)),
        ],
        compiler_params=pltpu.CompilerParams(collective_id=0),
    )(x, Wq, K_ext, V_ext, Wo)
